# Initial kernel scaffold; baseline (speedup 1.0000x reference)
import jax
import jax.numpy as jnp
from jax.experimental import pallas as pl


def kernel(x, A, W_l, W_r, att, bias):
    raise NotImplementedError("write your pallas kernel here")


# speedup vs baseline: 190.4165x; 190.4165x over previous
import jax
import jax.numpy as jnp
from jax.experimental import pallas as pl
from jax.experimental.pallas import tpu as pltpu


def _gat_kernel(x_ref, a_ref, wl_ref, wr_ref, att_ref, bias_ref, out_ref):
    n = a_ref.shape[0]
    c = wl_ref.shape[1]
    xg = x_ref[0]
    xl = jnp.dot(xg, wl_ref[...], preferred_element_type=jnp.float32)
    xr = jnp.dot(xg, wr_ref[...], preferred_element_type=jnp.float32)

    e = xl[:, None, :] + xr[None, :, :]
    e = jnp.where(e > 0, e, 0.2 * e)
    s = jax.lax.dot_general(
        e.reshape(n * n, c), att_ref[0],
        (((1,), (0,)), ((), ())),
        preferred_element_type=jnp.float32,
    ).reshape(n, n)

    row = jax.lax.broadcasted_iota(jnp.int32, (n, n), 0)
    col = jax.lax.broadcasted_iota(jnp.int32, (n, n), 1)
    diag = row == col
    valid = ((a_ref[...] != 0) & (~diag)) | diag
    s = jnp.where(valid, s, -jnp.inf)

    m = jnp.max(s, axis=0)
    p = jnp.exp(s - m[None, :])
    denom = jnp.sum(p, axis=0)
    alpha = p / denom[None, :]

    out = jax.lax.dot_general(
        alpha, xl, (((0,), (0,)), ((), ())),
        preferred_element_type=jnp.float32,
    )
    out_ref[0] = out + bias_ref[0][None, :]


def kernel(x, A, W_l, W_r, att, bias):
    B, T, N, F = x.shape
    H, C = att.shape
    assert H == 1
    G = B * T
    x3 = x.reshape(G, N, F)
    att2 = att.reshape(1, C)
    bias2 = bias.reshape(1, C)

    out = pl.pallas_call(
        _gat_kernel,
        grid=(G,),
        in_specs=[
            pl.BlockSpec((1, N, F), lambda g: (g, 0, 0)),
            pl.BlockSpec((N, N), lambda g: (0, 0)),
            pl.BlockSpec((F, C), lambda g: (0, 0)),
            pl.BlockSpec((F, C), lambda g: (0, 0)),
            pl.BlockSpec((1, C), lambda g: (0, 0)),
            pl.BlockSpec((1, C), lambda g: (0, 0)),
        ],
        out_specs=pl.BlockSpec((1, N, C), lambda g: (g, 0, 0)),
        out_shape=jax.ShapeDtypeStruct((G, N, C), jnp.float32),
        compiler_params=pltpu.CompilerParams(
            dimension_semantics=("arbitrary",),
        ),
    )(x3, A, W_l, W_r, att2, bias2)
    return out.reshape(B, T, N, C)
